# Initial kernel scaffold; baseline (speedup 1.0000x reference)
import jax
import jax.numpy as jnp
from jax.experimental import pallas as pl


def kernel(x, wires_p0, chans_p0, wires_p1, chans_p1, good_idx_01, ray_crossings_01):
    raise NotImplementedError("write your pallas kernel here")


# speedup vs baseline: 8.0841x; 8.0841x over previous
import functools

import jax
import jax.numpy as jnp
from jax import lax
from jax.experimental import pallas as pl
from jax.experimental.pallas import tpu as pltpu
from jax.experimental.pallas import tpu_sc as plsc

_F = 2
_C = 1536
_T = 1536
_NC = 1500
_NCP = 1504
_NWP = 480
_NFEAT = 15
_ROWW = _NC * _NFEAT
_ROWWP = _NCP * _NFEAT
_NWORK = 32
_TPW = _T // _NWORK
_TCHUNK = 16
_NGRP = _NCP // 16


def _sc_body(x3_h, cp0_h, cp1_h, wf0_h, wf1_h, g0_h, g1_h, r0_h, r1_h,
             out_h,
             xchunk, rowbuf, ch0v, ch1v, cb15v,
             g0v, g1v, r0v, r1v, cp0v, cp1v, wf0v, wf1v):
    cid = lax.axis_index("c")
    sid = lax.axis_index("s")
    wid = sid * 2 + cid

    pltpu.sync_copy(g0_h, g0v)
    pltpu.sync_copy(g1_h, g1v)
    pltpu.sync_copy(r0_h, r0v)
    pltpu.sync_copy(r1_h, r1v)
    pltpu.sync_copy(cp0_h, cp0v)
    pltpu.sync_copy(cp1_h, cp1v)
    pltpu.sync_copy(wf0_h, wf0v)
    pltpu.sync_copy(wf1_h, wf1v)

    zero16 = jnp.zeros((16,), jnp.float32)
    one16 = jnp.ones((16,), jnp.float32)
    iota16 = lax.iota(jnp.int32, 16)

    def init_grp(g, carry):
        base = g * 16
        cvec = iota16 + base
        cb = cvec * _NFEAT
        cb15v[pl.ds(base, 16)] = cb
        gv0 = g0v[pl.ds(base, 16)]
        gv1 = g1v[pl.ds(base, 16)]
        c0 = plsc.load_gather(cp0v, [gv0])
        c1 = plsc.load_gather(cp1v, [gv1])
        w0 = plsc.load_gather(wf0v, [gv0])
        w1 = plsc.load_gather(wf1v, [gv1])
        ch0v[pl.ds(base, 16)] = c0
        ch1v[pl.ds(base, 16)] = c1
        rr0 = r0v[pl.ds(base, 16)]
        rr1 = r1v[pl.ds(base, 16)]
        plsc.store_scatter(rowbuf, [cb + 2], w0)
        plsc.store_scatter(rowbuf, [cb + 3], c0.astype(jnp.float32))
        plsc.store_scatter(rowbuf, [cb + 4], zero16)
        plsc.store_scatter(rowbuf, [cb + 5], zero16)
        plsc.store_scatter(rowbuf, [cb + 8], w1)
        plsc.store_scatter(rowbuf, [cb + 9], c1.astype(jnp.float32))
        plsc.store_scatter(rowbuf, [cb + 10], zero16)
        plsc.store_scatter(rowbuf, [cb + 11], one16)
        plsc.store_scatter(rowbuf, [cb + 12], rr0)
        plsc.store_scatter(rowbuf, [cb + 13], rr1)
        return carry

    lax.fori_loop(0, _NGRP, init_grp, 0)

    t_base = wid * _TPW

    def chunk_body(ck, carry):
        t0 = t_base + ck * _TCHUNK
        pltpu.sync_copy(x3_h.at[:, pl.ds(t0, _TCHUNK)], xchunk)

        def tick_body(tt, carry2):
            t = t0 + tt
            ttvec = jnp.broadcast_to(tt, (16,))
            tf = jnp.broadcast_to(t.astype(jnp.float32), (16,))

            def grp(g, carry3):
                base = g * 16
                cb = cb15v[pl.ds(base, 16)]
                c0 = ch0v[pl.ds(base, 16)]
                c1 = ch1v[pl.ds(base, 16)]
                v00 = plsc.load_gather(xchunk, [c0, ttvec])
                v01 = plsc.load_gather(xchunk, [c0 + _C, ttvec])
                v10 = plsc.load_gather(xchunk, [c1, ttvec])
                v11 = plsc.load_gather(xchunk, [c1 + _C, ttvec])
                plsc.store_scatter(rowbuf, [cb], v00)
                plsc.store_scatter(rowbuf, [cb + 1], v01)
                plsc.store_scatter(rowbuf, [cb + 6], v10)
                plsc.store_scatter(rowbuf, [cb + 7], v11)
                plsc.store_scatter(rowbuf, [cb + 14], tf)
                return carry3

            lax.fori_loop(0, _NGRP, grp, 0)
            pltpu.sync_copy(rowbuf.at[pl.ds(0, _ROWW)], out_h.at[t])
            return carry2

        lax.fori_loop(0, _TCHUNK, tick_body, 0)
        return carry

    lax.fori_loop(0, _TPW // _TCHUNK, chunk_body, 0)


def kernel(x, wires_p0, chans_p0, wires_p1, chans_p1, good_idx_01,
           ray_crossings_01):
    nbatch, nfb, _, nticks = x.shape
    x3 = x.reshape(_F * _C, _T)
    g0 = jnp.pad(good_idx_01[:, 0].astype(jnp.int32), (0, _NCP - _NC))
    g1 = jnp.pad(good_idx_01[:, 1].astype(jnp.int32), (0, _NCP - _NC))
    r0 = jnp.pad(ray_crossings_01[:, 0].astype(jnp.float32), (0, _NCP - _NC))
    r1 = jnp.pad(ray_crossings_01[:, 1].astype(jnp.float32), (0, _NCP - _NC))
    nw0 = chans_p0.shape[0]
    nw1 = chans_p1.shape[0]
    cp0 = jnp.pad(chans_p0.astype(jnp.int32), (0, _NWP - nw0))
    cp1 = jnp.pad(chans_p1.astype(jnp.int32), (0, _NWP - nw1))
    wf0 = jnp.pad(wires_p0.astype(jnp.float32), (0, _NWP - nw0))
    wf1 = jnp.pad(wires_p1.astype(jnp.float32), (0, _NWP - nw1))

    mesh = plsc.VectorSubcoreMesh(core_axis_name="c", subcore_axis_name="s")
    run = pl.kernel(
        _sc_body,
        out_type=jax.ShapeDtypeStruct((_T, _ROWW), jnp.float32),
        mesh=mesh,
        compiler_params=pltpu.CompilerParams(use_tc_tiling_on_sc=False,
                                             needs_layout_passes=False),
        scratch_types=[
            pltpu.VMEM((_F * _C, _TCHUNK), jnp.float32),
            pltpu.VMEM((_ROWWP,), jnp.float32),
            pltpu.VMEM((_NCP,), jnp.int32),
            pltpu.VMEM((_NCP,), jnp.int32),
            pltpu.VMEM((_NCP,), jnp.int32),
            pltpu.VMEM((_NCP,), jnp.int32),
            pltpu.VMEM((_NCP,), jnp.int32),
            pltpu.VMEM((_NCP,), jnp.float32),
            pltpu.VMEM((_NCP,), jnp.float32),
            pltpu.VMEM((_NWP,), jnp.int32),
            pltpu.VMEM((_NWP,), jnp.int32),
            pltpu.VMEM((_NWP,), jnp.float32),
            pltpu.VMEM((_NWP,), jnp.float32),
        ],
    )
    out2d = run(x3, cp0, cp1, wf0, wf1, g0, g1, r0, r1)
    return out2d.reshape(nbatch, nticks, _NC, _NFEAT)
